# baseline (speedup 1.0000x reference)
import dataclasses
import functools

import jax
import jax.numpy as jnp
from jax import lax
from jax.experimental import pallas as pl
from jax.experimental.pallas import tpu as pltpu
from jax.experimental.pallas import tpu_sc as plsc

NUM_CORES = 2
NUM_SUBCORES = 16
NUM_WORKERS = NUM_CORES * NUM_SUBCORES
LANES = 16
EMB = 64
BC = 32768
GCHUNK = 128
SCCHUNK = 512
SC_NCH = 49
SC_CPW = SC_NCH * SCCHUNK
SC_COLS = NUM_WORKERS * SC_CPW


def _sc_params():
    cp = pltpu.CompilerParams()
    if "needs_layout_passes" in pltpu.CompilerParams.__dataclass_fields__:
        cp = dataclasses.replace(cp, needs_layout_passes=False)
    return cp


def _tc_scan(xT, w, blk_lo, blk_hi):
    V = xT.shape[1]

    def body(xT_ref, w_ref, s_ref):
        s_ref[...] = jnp.sum(xT_ref[...] * w_ref[...], axis=0)

    return pl.pallas_call(
        body,
        grid=(blk_hi - blk_lo,),
        in_specs=[
            pl.BlockSpec((EMB, BC), lambda j: (0, blk_lo + j)),
            pl.BlockSpec((EMB, 1), lambda j: (0, 0)),
        ],
        out_specs=pl.BlockSpec((BC,), lambda j: (blk_lo + j,)),
        out_shape=jax.ShapeDtypeStruct((V,), jnp.float32),
    )(xT, w)


def _sc_scan_item(iT, wi):
    mesh = plsc.VectorSubcoreMesh(core_axis_name="c", subcore_axis_name="s")

    @functools.partial(
        pl.kernel,
        out_type=jax.ShapeDtypeStruct((SC_COLS,), jnp.float32),
        mesh=mesh,
        compiler_params=_sc_params(),
        scratch_types=[
            pltpu.VMEM((EMB, SCCHUNK), jnp.float32),
            pltpu.VMEM((EMB, SCCHUNK), jnp.float32),
            pltpu.VMEM((EMB,), jnp.float32),
            pltpu.VMEM((SCCHUNK,), jnp.float32),
            pltpu.SemaphoreType.DMA,
            pltpu.SemaphoreType.DMA,
        ],
    )
    def k(iT_h, wi_h, out_h, buf0, buf1, w_v, sc_v, sem0, sem1):
        wid = lax.axis_index("s") * NUM_CORES + lax.axis_index("c")
        base = wid * SC_CPW
        pltpu.sync_copy(wi_h, w_v)
        wch = [w_v[pl.ds(LANES * j, LANES)] for j in range(EMB // LANES)]

        def start(k_, buf, sem):
            return pltpu.async_copy(
                iT_h.at[:, pl.ds(base + k_ * SCCHUNK, SCCHUNK)], buf, sem)

        def wait(buf, sem):
            pltpu.make_async_copy(
                iT_h.at[:, pl.ds(base, SCCHUNK)], buf, sem).wait()

        def compute(k_, buf):
            @pl.loop(0, SCCHUNK // LANES)
            def _(g):
                sl = pl.ds(g * LANES, LANES)
                accs = [buf[a, sl] * wch[0][a] for a in range(4)]
                for d in range(4, EMB):
                    a = d & 3
                    accs[a] = accs[a] + buf[d, sl] * wch[d // LANES][d % LANES]
                sc_v[sl] = (accs[0] + accs[1]) + (accs[2] + accs[3])
            pltpu.sync_copy(sc_v, out_h.at[pl.ds(base + k_ * SCCHUNK, SCCHUNK)])

        start(0, buf0, sem0)

        @pl.loop(0, SC_NCH - 1, step=2)
        def _(kk):
            start(kk + 1, buf1, sem1)
            wait(buf0, sem0)
            compute(kk, buf0)
            start(kk + 2, buf0, sem0)
            wait(buf1, sem1)
            compute(kk + 1, buf1)

        wait(buf0, sem0)
        compute(SC_NCH - 1, buf0)

    return k(iT, wi)


def _sc_gather_sigmoid(user, item, u_score, i_main, i_tail, bias16):
    B = user.shape[0]
    bpw = B // NUM_WORKERS
    n_chunks = bpw // GCHUNK
    mesh = plsc.VectorSubcoreMesh(core_axis_name="c", subcore_axis_name="s")

    @functools.partial(
        pl.kernel,
        out_type=jax.ShapeDtypeStruct((B,), jnp.float32),
        mesh=mesh,
        compiler_params=_sc_params(),
        scratch_types=[
            pltpu.VMEM((bpw,), jnp.int32),
            pltpu.VMEM((bpw,), jnp.int32),
            pltpu.VMEM((bpw,), jnp.int32),
            pltpu.VMEM((bpw,), jnp.float32),
            pltpu.VMEM((bpw,), jnp.float32),
            pltpu.VMEM((bpw,), jnp.float32),
            pltpu.VMEM((LANES,), jnp.float32),
            pltpu.SemaphoreType.DMA,
            pltpu.SemaphoreType.DMA,
            pltpu.SemaphoreType.DMA,
        ],
    )
    def k(user_h, item_h, us_h, im_h, it_h, b_h, out_h,
          uidx_v, iidx_v, midx_v, uval_v, mval_v, tval_v, b_v,
          sem_u, sem_m, sem_t):
        wid = lax.axis_index("s") * NUM_CORES + lax.axis_index("c")
        base = wid * bpw
        pltpu.sync_copy(b_h, b_v)
        pltpu.sync_copy(user_h.at[pl.ds(base, bpw)], uidx_v)
        pltpu.sync_copy(item_h.at[pl.ds(base, bpw)], iidx_v)

        @pl.loop(0, bpw, step=LANES)
        def _(i):
            sl = pl.ds(i, LANES)
            ii = iidx_v[sl]
            midx_v[sl] = jnp.where(ii < SC_COLS, ii, ii - SC_COLS)

        copies = []
        for c in range(n_chunks):
            sl = pl.ds(c * GCHUNK, GCHUNK)
            copies.append(pltpu.async_copy(
                us_h.at[uidx_v.at[sl]], uval_v.at[sl], sem_u))
            copies.append(pltpu.async_copy(
                im_h.at[midx_v.at[sl]], mval_v.at[sl], sem_m))
            copies.append(pltpu.async_copy(
                it_h.at[iidx_v.at[sl]], tval_v.at[sl], sem_t))
        for cpy in copies:
            cpy.wait()
        bias = b_v[pl.ds(0, LANES)]

        @pl.loop(0, bpw, step=LANES)
        def _(i):
            sl = pl.ds(i, LANES)
            ival = jnp.where(iidx_v[sl] >= SC_COLS, tval_v[sl], mval_v[sl])
            x = uval_v[sl] + ival + bias
            uval_v[sl] = 1.0 / (1.0 + jnp.exp(-x))

        pltpu.sync_copy(uval_v, out_h.at[pl.ds(base, bpw)])

    return k(user, item, u_score, i_main, i_tail, bias16)


def kernel(user, item, user_emb, item_emb, fc_w, fc_b):
    w = fc_w.reshape(-1).astype(jnp.float32)
    wu = w[:EMB].reshape(EMB, 1)
    wi = w[EMB:].reshape(EMB, 1)
    uT = user_emb.T
    iT = item_emb.T
    n_blocks = pl.cdiv(uT.shape[1], BC)
    u_score = _tc_scan(uT, wu, 0, n_blocks)
    tail_blk = SC_COLS // BC
    i_tail = _tc_scan(iT, wi, tail_blk, n_blocks)
    i_main = _sc_scan_item(iT, w[EMB:])
    bias16 = jnp.broadcast_to(fc_b.astype(jnp.float32), (LANES,))
    out = _sc_gather_sigmoid(user.astype(jnp.int32), item.astype(jnp.int32),
                             u_score, i_main, i_tail, bias16)
    return out.reshape(-1, 1)


# speedup vs baseline: 1.0358x; 1.0358x over previous
import dataclasses
import functools

import jax
import jax.numpy as jnp
from jax import lax
from jax.experimental import pallas as pl
from jax.experimental.pallas import tpu as pltpu
from jax.experimental.pallas import tpu_sc as plsc

NUM_CORES = 2
NUM_SUBCORES = 16
NUM_WORKERS = NUM_CORES * NUM_SUBCORES
LANES = 16
EMB = 64
BC = 32768
GCHUNK = 128


def _tc_scan_scores(uT, iT, wu, wi):
    V = uT.shape[1]
    grid = (pl.cdiv(V, BC),)

    def body(uT_ref, iT_ref, wu_ref, wi_ref, us_ref, is_ref):
        us_ref[...] = jnp.sum(uT_ref[...] * wu_ref[...], axis=0)
        is_ref[...] = jnp.sum(iT_ref[...] * wi_ref[...], axis=0)

    return pl.pallas_call(
        body,
        grid=grid,
        in_specs=[
            pl.BlockSpec((EMB, BC), lambda j: (0, j)),
            pl.BlockSpec((EMB, BC), lambda j: (0, j)),
            pl.BlockSpec((EMB, 1), lambda j: (0, 0)),
            pl.BlockSpec((EMB, 1), lambda j: (0, 0)),
        ],
        out_specs=[
            pl.BlockSpec((BC,), lambda j: (j,)),
            pl.BlockSpec((BC,), lambda j: (j,)),
        ],
        out_shape=[
            jax.ShapeDtypeStruct((V,), jnp.float32),
            jax.ShapeDtypeStruct((V,), jnp.float32),
        ],
    )(uT, iT, wu, wi)


def _sc_gather_sigmoid(user, item, u_score, i_score, bias16):
    B = user.shape[0]
    bpw = B // NUM_WORKERS
    n_chunks = bpw // GCHUNK
    mesh = plsc.VectorSubcoreMesh(core_axis_name="c", subcore_axis_name="s")
    cp = pltpu.CompilerParams()
    if "needs_layout_passes" in pltpu.CompilerParams.__dataclass_fields__:
        cp = dataclasses.replace(cp, needs_layout_passes=False)

    @functools.partial(
        pl.kernel,
        out_type=jax.ShapeDtypeStruct((B,), jnp.float32),
        mesh=mesh,
        compiler_params=cp,
        scratch_types=[
            pltpu.VMEM((bpw,), jnp.int32),
            pltpu.VMEM((bpw,), jnp.int32),
            pltpu.VMEM((bpw,), jnp.float32),
            pltpu.VMEM((bpw,), jnp.float32),
            pltpu.VMEM((LANES,), jnp.float32),
            pltpu.SemaphoreType.DMA,
            pltpu.SemaphoreType.DMA,
        ],
    )
    def k(user_h, item_h, us_h, is_h, b_h, out_h,
          uidx_v, iidx_v, uval_v, ival_v, b_v, sem_u, sem_i):
        wid = lax.axis_index("s") * NUM_CORES + lax.axis_index("c")
        base = wid * bpw
        pltpu.sync_copy(b_h, b_v)
        pltpu.sync_copy(user_h.at[pl.ds(base, bpw)], uidx_v)
        pltpu.sync_copy(item_h.at[pl.ds(base, bpw)], iidx_v)
        copies = []
        for c in range(n_chunks):
            sl = pl.ds(c * GCHUNK, GCHUNK)
            copies.append(pltpu.async_copy(
                us_h.at[uidx_v.at[sl]], uval_v.at[sl], sem_u))
            copies.append(pltpu.async_copy(
                is_h.at[iidx_v.at[sl]], ival_v.at[sl], sem_i))
        for cpy in copies:
            cpy.wait()
        bias = b_v[pl.ds(0, LANES)]

        @pl.loop(0, bpw, step=LANES)
        def _(i):
            sl = pl.ds(i, LANES)
            x = uval_v[sl] + ival_v[sl] + bias
            uval_v[sl] = 1.0 / (1.0 + jnp.exp(-x))

        pltpu.sync_copy(uval_v, out_h.at[pl.ds(base, bpw)])

    return k(user, item, u_score, i_score, bias16)


def kernel(user, item, user_emb, item_emb, fc_w, fc_b):
    w = fc_w.reshape(-1).astype(jnp.float32)
    wu = w[:EMB].reshape(EMB, 1)
    wi = w[EMB:].reshape(EMB, 1)
    u_score, i_score = _tc_scan_scores(user_emb.T, item_emb.T, wu, wi)
    bias16 = jnp.broadcast_to(fc_b.astype(jnp.float32), (LANES,))
    out = _sc_gather_sigmoid(user.astype(jnp.int32), item.astype(jnp.int32),
                             u_score, i_score, bias16)
    return out.reshape(-1, 1)
